# baseline (speedup 1.0000x reference)
import functools

import jax
import jax.numpy as jnp
from jax import lax
from jax.experimental import pallas as pl
from jax.experimental.pallas import tpu as pltpu
from jax.experimental.pallas import tpu_sc as plsc

_NUM_WORKERS = 32
_CHUNK = 8
_NBUF = 8


@functools.lru_cache(maxsize=None)
def _build_gather(V, D, n_total):
    n_per_w = n_total // _NUM_WORKERS
    n_chunks = n_per_w // _CHUNK
    mesh = plsc.VectorSubcoreMesh(core_axis_name="c", subcore_axis_name="s")

    @functools.partial(
        pl.kernel,
        mesh=mesh,
        out_type=jax.ShapeDtypeStruct((n_total, D), jnp.float32),
        scratch_types=(
            [pltpu.VMEM((n_chunks, _CHUNK), jnp.int32)]
            + [pltpu.VMEM((_CHUNK, D), jnp.float32)] * _NBUF
            + [pltpu.SemaphoreType.DMA] * (2 * _NBUF)
        ),
    )
    def gather_kernel(table_hbm, idx_hbm, out_hbm, idx_v, *rest):
        bufs = tuple(zip(rest[:_NBUF], rest[_NBUF:2 * _NBUF],
                         rest[2 * _NBUF:]))
        wid = lax.axis_index("s") * 2 + lax.axis_index("c")
        pltpu.sync_copy(idx_hbm.at[wid], idx_v)
        base = wid * n_per_w

        def out_at(j):
            return out_hbm.at[pl.ds(base + j * _CHUNK, _CHUNK)]

        for b, (buf, gsem, wsem) in enumerate(bufs):
            pltpu.async_copy(table_hbm.at[idx_v.at[b]], buf, gsem)

        def body(i, carry):
            for b, (buf, gsem, wsem) in enumerate(bufs):
                j = _NBUF * i + b
                pltpu.make_async_copy(table_hbm.at[idx_v.at[j]], buf, gsem).wait()
                pltpu.async_copy(buf, out_at(j), wsem)
            for b, (buf, gsem, wsem) in enumerate(bufs):
                j = _NBUF * i + b
                pltpu.make_async_copy(buf, out_at(j), wsem).wait()
                pltpu.async_copy(table_hbm.at[idx_v.at[j + _NBUF]], buf, gsem)
            return carry

        lax.fori_loop(0, n_chunks // _NBUF - 1, body, 0)

        for b, (buf, gsem, wsem) in enumerate(bufs):
            j = n_chunks - _NBUF + b
            pltpu.make_async_copy(table_hbm.at[idx_v.at[j]], buf, gsem).wait()
            pltpu.async_copy(buf, out_at(j), wsem)
        for b, (buf, gsem, wsem) in enumerate(bufs):
            j = n_chunks - _NBUF + b
            pltpu.make_async_copy(buf, out_at(j), wsem).wait()

    return gather_kernel


def kernel(input_ids, embed_tokens):
    B, S = input_ids.shape
    V, D = embed_tokens.shape
    n_total = B * S
    ids = input_ids.reshape(_NUM_WORKERS, (n_total // _NUM_WORKERS) // _CHUNK,
                            _CHUNK).astype(jnp.int32)
    out = _build_gather(V, D, n_total)(embed_tokens, ids)
    return out.reshape(B, S, D)


# speedup vs baseline: 1.0351x; 1.0351x over previous
import functools

import jax
import jax.numpy as jnp
from jax import lax
from jax.experimental import pallas as pl
from jax.experimental.pallas import tpu as pltpu
from jax.experimental.pallas import tpu_sc as plsc

_NUM_WORKERS = 32
_CHUNK = 8
_NBUF = 8


@functools.lru_cache(maxsize=None)
def _build_gather(V, D, n_total):
    n_per_w = n_total // _NUM_WORKERS
    n_chunks = n_per_w // _CHUNK
    mesh = plsc.VectorSubcoreMesh(core_axis_name="c", subcore_axis_name="s")

    @functools.partial(
        pl.kernel,
        mesh=mesh,
        out_type=jax.ShapeDtypeStruct((n_total, D), jnp.float32),
        scratch_types=(
            [pltpu.VMEM((n_chunks, _CHUNK), jnp.int32)]
            + [pltpu.VMEM((_CHUNK, D), jnp.float32)] * _NBUF
            + [pltpu.SemaphoreType.DMA] * _NBUF
        ),
    )
    def gather_kernel(table_hbm, idx_hbm, out_hbm, idx_v, *rest):
        bufs = tuple(zip(rest[:_NBUF], rest[_NBUF:]))
        wid = lax.axis_index("s") * 2 + lax.axis_index("c")
        pltpu.sync_copy(idx_hbm.at[wid], idx_v)
        base = wid * n_per_w

        for b, (buf, sem) in enumerate(bufs):
            pltpu.async_copy(table_hbm.at[idx_v.at[b]], buf, sem)

        def body(i, carry):
            for b, (buf, sem) in enumerate(bufs):
                j = _NBUF * i + b
                pltpu.make_async_copy(table_hbm.at[idx_v.at[j]], buf, sem).wait()
                pltpu.sync_copy(buf, out_hbm.at[pl.ds(base + j * _CHUNK, _CHUNK)])
                pltpu.async_copy(table_hbm.at[idx_v.at[j + _NBUF]], buf, sem)
            return carry

        lax.fori_loop(0, n_chunks // _NBUF - 1, body, 0)

        for b, (buf, sem) in enumerate(bufs):
            j = n_chunks - _NBUF + b
            pltpu.make_async_copy(table_hbm.at[idx_v.at[j]], buf, sem).wait()
            pltpu.sync_copy(buf, out_hbm.at[pl.ds(base + j * _CHUNK, _CHUNK)])

    return gather_kernel


def kernel(input_ids, embed_tokens):
    B, S = input_ids.shape
    V, D = embed_tokens.shape
    n_total = B * S
    ids = input_ids.reshape(_NUM_WORKERS, (n_total // _NUM_WORKERS) // _CHUNK,
                            _CHUNK).astype(jnp.int32)
    out = _build_gather(V, D, n_total)(embed_tokens, ids)
    return out.reshape(B, S, D)
